# baseline (speedup 1.0000x reference)
import functools

import jax
import jax.numpy as jnp
from jax import lax
from jax.experimental import pallas as pl
from jax.experimental.pallas import tpu as pltpu
from jax.experimental.pallas import tpu_sc as plsc

S = 4096
T = 200
D = 64
NC, NS = 2, 16
NW = NC * NS
C = 128
UNITS = (S // C) * T
UPW = UNITS // NW
SB = S // C

_mesh = plsc.VectorSubcoreMesh(core_axis_name="c", subcore_axis_name="s")


@functools.partial(
    pl.kernel,
    mesh=_mesh,
    out_type=jax.ShapeDtypeStruct((T, S, 128), jnp.float32),
    scratch_types=[
        pltpu.VMEM((4, C), jnp.int32),
        pltpu.VMEM((2, C, 128), jnp.float32),
        pltpu.SemaphoreType.DMA((4,)),
        pltpu.SemaphoreType.DMA((2,)),
        pltpu.SemaphoreType.DMA((2,)),
    ],
    compiler_params=pltpu.CompilerParams(
        use_tc_tiling_on_sc=True, needs_layout_passes=False),
)
def _gather_kernel(xt_hbm, tbl_hbm, p_hbm, idx_v, rows_v, isem, gsem, wsem):
    wid = lax.axis_index("s") * NC + lax.axis_index("c")
    u0 = wid * UPW

    def stage(i):
        u = u0 + i
        t = u // SB
        s0 = (u % SB) * C
        return pltpu.make_async_copy(
            xt_hbm.at[t, pl.ds(s0, C)], idx_v.at[i % 4], isem.at[i % 4])

    def gather(i):
        return pltpu.make_async_copy(
            tbl_hbm.at[idx_v.at[i % 4]], rows_v.at[i % 2], gsem.at[i % 2])

    def write(i):
        u = u0 + i
        t = u // SB
        s0 = (u % SB) * C
        return pltpu.make_async_copy(
            rows_v.at[i % 2],
            p_hbm.at[t, pl.ds(s0, C), :], wsem.at[i % 2])

    for i in range(4):
        stage(i).start()
    stage(0).wait()
    gather(0).start()
    stage(1).wait()
    gather(1).start()

    def body(i, carry):
        gather(i).wait()

        @pl.when(i >= 2)
        def _():
            write(i - 2).wait()

        write(i).start()

        @pl.when(i + 4 < UPW)
        def _():
            stage(i + 4).start()

        @pl.when(i + 2 < UPW)
        def _():
            stage(i + 2).wait()
            gather(i + 2).start()

        return carry

    lax.fori_loop(0, UPW, body, 0)
    write(UPW - 2).wait()
    write(UPW - 1).wait()


def kernel(x, table):
    xt = x.T
    tblpad = jnp.pad(table, ((0, 0), (0, 64)))
    p = _gather_kernel(xt, tblpad)
    return jnp.swapaxes(p[:, :, :D], 0, 1)
